# baseline (speedup 1.0000x reference)
import functools

import jax
import jax.numpy as jnp
from jax import lax
from jax.experimental import pallas as pl
from jax.experimental.pallas import tpu as pltpu
from jax.experimental.pallas import tpu_sc as plsc

N = 10000
E = 320000
D = 128

NC = 2
NS = 16
CH = 128
NCH = 80
TPW = NCH * CH
E_PAD = NC * NS * TPW
N_PAD = 10240
RPT = N_PAD // NS


PH = NCH // 2


def _sc_agg_body(h_hbm, src_hbm, dst_hbm, zeros_hbm, out_hbm,
                 idx_s, idx_d, rows, agg_sh, sg0, sg1, ss0, ss1, si):
    cid = lax.axis_index("c")
    sid = lax.axis_index("s")
    tile = (1 - cid) * NS + sid
    sg = [sg0, sg1]
    ss = [ss0, ss1]

    r0 = sid * RPT
    for k in range(RPT // 128):
        pltpu.sync_copy(zeros_hbm, agg_sh.at[pl.ds(r0 + k * 128, 128)])

    plsc.subcore_barrier()

    def g_start(j, b):
        pltpu.async_copy(h_hbm.at[idx_s.at[j]], rows.at[b], sg[b])

    def g_wait(b):
        pltpu.make_async_copy(h_hbm.at[idx_s.at[0]], rows.at[b], sg[b]).wait()

    def s_start(i, b):
        pltpu.async_copy(rows.at[b], agg_sh.at[idx_d.at[i]], ss[b], add=True)

    def s_wait(b):
        pltpu.make_async_copy(rows.at[b], agg_sh.at[idx_d.at[0]], ss[b]).wait()

    for p in range(2):
        base = tile * NCH + p * PH
        pltpu.async_copy(src_hbm.at[pl.ds(base, PH)], idx_s, si)
        pltpu.async_copy(dst_hbm.at[pl.ds(base, PH)], idx_d, si)
        pltpu.make_async_copy(src_hbm.at[pl.ds(base, PH)], idx_s, si).wait()
        pltpu.make_async_copy(dst_hbm.at[pl.ds(base, PH)], idx_d, si).wait()

        g_start(0, 0)
        g_wait(0); s_start(0, 0); g_start(1, 1)

        def pair(j, carry):
            c = 2 * j + 1
            g_wait(1); s_start(c, 1); s_wait(0); g_start(c + 1, 0)
            g_wait(0); s_start(c + 1, 0); s_wait(1); g_start(c + 2, 1)
            return carry

        lax.fori_loop(0, (PH - 2) // 2, pair, 0)

        g_wait(1); s_start(PH - 1, 1)
        s_wait(0)
        s_wait(1)

    plsc.subcore_barrier()

    pltpu.sync_copy(agg_sh.at[pl.ds(r0, RPT)],
                    out_hbm.at[cid].at[pl.ds(r0, RPT)])


_sc_agg = functools.partial(
    pl.kernel,
    out_type=jax.ShapeDtypeStruct((NC, N_PAD, D), jnp.float32),
    mesh=plsc.VectorSubcoreMesh(
        core_axis_name="c", subcore_axis_name="s", num_cores=NC,
        num_subcores=NS),
    scratch_types=[
        pltpu.VMEM((PH, CH), jnp.int32),
        pltpu.VMEM((PH, CH), jnp.int32),
        pltpu.VMEM((2, CH, D), jnp.float32),
        pltpu.VMEM_SHARED((N_PAD, D), jnp.float32),
        pltpu.SemaphoreType.DMA, pltpu.SemaphoreType.DMA,
        pltpu.SemaphoreType.DMA, pltpu.SemaphoreType.DMA,
        pltpu.SemaphoreType.DMA,
    ],
)(_sc_agg_body)


BLK = 1000


def _mlp1_body(h_ref, p_ref, wa_ref, ba_ref, wb_ref, bb_ref, o_ref):
    z = h_ref[...] + p_ref[0] + p_ref[1]
    z = jnp.maximum(
        jnp.dot(z, wa_ref[...], preferred_element_type=jnp.float32)
        + ba_ref[...], 0.0)
    z = jnp.dot(z, wb_ref[...], preferred_element_type=jnp.float32) \
        + bb_ref[...]
    o_ref[...] = jnp.maximum(z, 0.0)


def _mlp2_body(h_ref, p_ref, wa_ref, ba_ref, wb_ref, bb_ref,
               wl_ref, bl_ref, o_ref, acc_ref):
    i = pl.program_id(0)
    z = h_ref[...] + p_ref[0] + p_ref[1]
    z = jnp.maximum(
        jnp.dot(z, wa_ref[...], preferred_element_type=jnp.float32)
        + ba_ref[...], 0.0)
    z = jnp.dot(z, wb_ref[...], preferred_element_type=jnp.float32) \
        + bb_ref[...]
    col = jnp.sum(jnp.maximum(z, 0.0), axis=0, keepdims=True)

    @pl.when(i == 0)
    def _():
        acc_ref[...] = col

    @pl.when(i > 0)
    def _():
        acc_ref[...] = acc_ref[...] + col

    @pl.when(i == pl.num_programs(0) - 1)
    def _():
        o_ref[...] = jnp.dot(acc_ref[...], wl_ref[...],
                             preferred_element_type=jnp.float32) + bl_ref[...]


_w_spec = pl.BlockSpec((D, D), lambda i: (0, 0))
_b_spec = pl.BlockSpec((1, D), lambda i: (0, 0))
_h_spec = pl.BlockSpec((BLK, D), lambda i: (i, 0))
_p_spec = pl.BlockSpec((NC, BLK, D), lambda i: (0, i, 0))

_mlp1 = pl.pallas_call(
    _mlp1_body,
    grid=(N // BLK,),
    in_specs=[_h_spec, _p_spec, _w_spec, _b_spec, _w_spec, _b_spec],
    out_specs=_h_spec,
    out_shape=jax.ShapeDtypeStruct((N, D), jnp.float32),
)

_mlp2 = pl.pallas_call(
    _mlp2_body,
    grid=(N // BLK,),
    in_specs=[_h_spec, _p_spec, _w_spec, _b_spec, _w_spec, _b_spec,
              _w_spec, _b_spec],
    out_specs=pl.BlockSpec((1, D), lambda i: (0, 0)),
    out_shape=jax.ShapeDtypeStruct((1, D), jnp.float32),
    scratch_shapes=[pltpu.VMEM((1, D), jnp.float32)],
)


def kernel(x, edge_index, W1a, b1a, W1b, b1b, W2a, b2a, W2b, b2b, Wlin, blin):
    pad = E_PAD - E
    src = jnp.concatenate(
        [edge_index[0], jnp.zeros((pad,), jnp.int32)]).reshape(E_PAD // CH, CH)
    dst = jnp.concatenate(
        [edge_index[1], jnp.full((pad,), N, jnp.int32)]).reshape(E_PAD // CH, CH)
    zeros = jnp.zeros((128, D), jnp.float32)
    b1a2, b1b2 = b1a.reshape(1, D), b1b.reshape(1, D)
    b2a2, b2b2 = b2a.reshape(1, D), b2b.reshape(1, D)
    blin2 = blin.reshape(1, D)

    p1 = _sc_agg(x, src, dst, zeros)
    h1 = _mlp1(x, p1, W1a, b1a2, W1b, b1b2)
    p2 = _sc_agg(h1, src, dst, zeros)
    return _mlp2(h1, p2, W2a, b2a2, W2b, b2b2, Wlin, blin2)


# speedup vs baseline: 1.0010x; 1.0010x over previous
import functools

import jax
import jax.numpy as jnp
from jax import lax
from jax.experimental import pallas as pl
from jax.experimental.pallas import tpu as pltpu
from jax.experimental.pallas import tpu_sc as plsc

N = 10000
E = 320000
D = 128

NC = 2
NS = 16
CH = 128
NCH = 80
TPW = NCH * CH
E_PAD = NC * NS * TPW
N_PAD = 10240
RPT = N_PAD // NS


PH = NCH // 2


def _sc_agg_body(h_hbm, src_hbm, dst_hbm, zeros_hbm, out_hbm,
                 idx_s, idx_d, rows, agg_sh, sg0, sg1, ss0, ss1, si):
    cid = lax.axis_index("c")
    sid = lax.axis_index("s")
    tile = cid * NS + sid
    sg = [sg0, sg1]
    ss = [ss0, ss1]

    r0 = sid * RPT
    for k in range(RPT // 128):
        pltpu.sync_copy(zeros_hbm, agg_sh.at[pl.ds(r0 + k * 128, 128)])

    plsc.subcore_barrier()

    def g_start(j, b):
        pltpu.async_copy(h_hbm.at[idx_s.at[j]], rows.at[b], sg[b])

    def g_wait(b):
        pltpu.make_async_copy(h_hbm.at[idx_s.at[0]], rows.at[b], sg[b]).wait()

    def s_start(i, b):
        pltpu.async_copy(rows.at[b], agg_sh.at[idx_d.at[i]], ss[b], add=True)

    def s_wait(b):
        pltpu.make_async_copy(rows.at[b], agg_sh.at[idx_d.at[0]], ss[b]).wait()

    for p in range(2):
        base = tile * NCH + p * PH
        pltpu.async_copy(src_hbm.at[pl.ds(base, PH)], idx_s, si)
        pltpu.async_copy(dst_hbm.at[pl.ds(base, PH)], idx_d, si)
        pltpu.make_async_copy(src_hbm.at[pl.ds(base, PH)], idx_s, si).wait()
        pltpu.make_async_copy(dst_hbm.at[pl.ds(base, PH)], idx_d, si).wait()

        g_start(0, 0)
        g_wait(0); s_start(0, 0); g_start(1, 1)

        def pair(j, carry):
            c = 2 * j + 1
            g_wait(1); s_start(c, 1); s_wait(0); g_start(c + 1, 0)
            g_wait(0); s_start(c + 1, 0); s_wait(1); g_start(c + 2, 1)
            return carry

        lax.fori_loop(0, (PH - 2) // 2, pair, 0)

        g_wait(1); s_start(PH - 1, 1)
        s_wait(0)
        s_wait(1)

    plsc.subcore_barrier()

    pltpu.sync_copy(agg_sh.at[pl.ds(r0, RPT)],
                    out_hbm.at[cid].at[pl.ds(r0, RPT)])


_sc_agg = functools.partial(
    pl.kernel,
    out_type=jax.ShapeDtypeStruct((NC, N_PAD, D), jnp.float32),
    mesh=plsc.VectorSubcoreMesh(
        core_axis_name="c", subcore_axis_name="s", num_cores=NC,
        num_subcores=NS),
    scratch_types=[
        pltpu.VMEM((PH, CH), jnp.int32),
        pltpu.VMEM((PH, CH), jnp.int32),
        pltpu.VMEM((2, CH, D), jnp.float32),
        pltpu.VMEM_SHARED((N_PAD, D), jnp.float32),
        pltpu.SemaphoreType.DMA, pltpu.SemaphoreType.DMA,
        pltpu.SemaphoreType.DMA, pltpu.SemaphoreType.DMA,
        pltpu.SemaphoreType.DMA,
    ],
)(_sc_agg_body)


BLK = 1000


def _mlp1_body(h_ref, p_ref, wa_ref, ba_ref, wb_ref, bb_ref, o_ref):
    z = h_ref[...] + p_ref[0] + p_ref[1]
    z = jnp.maximum(
        jnp.dot(z, wa_ref[...], preferred_element_type=jnp.float32)
        + ba_ref[...], 0.0)
    z = jnp.dot(z, wb_ref[...], preferred_element_type=jnp.float32) \
        + bb_ref[...]
    o_ref[...] = jnp.maximum(z, 0.0)


def _mlp2_body(h_ref, p_ref, wa_ref, ba_ref, wb_ref, bb_ref,
               wl_ref, bl_ref, o_ref, acc_ref):
    i = pl.program_id(0)
    z = h_ref[...] + p_ref[0] + p_ref[1]
    z = jnp.maximum(
        jnp.dot(z, wa_ref[...], preferred_element_type=jnp.float32)
        + ba_ref[...], 0.0)
    z = jnp.dot(z, wb_ref[...], preferred_element_type=jnp.float32) \
        + bb_ref[...]
    col = jnp.sum(jnp.maximum(z, 0.0), axis=0, keepdims=True)

    @pl.when(i == 0)
    def _():
        acc_ref[...] = col

    @pl.when(i > 0)
    def _():
        acc_ref[...] = acc_ref[...] + col

    @pl.when(i == pl.num_programs(0) - 1)
    def _():
        o_ref[...] = jnp.dot(acc_ref[...], wl_ref[...],
                             preferred_element_type=jnp.float32) + bl_ref[...]


_w_spec = pl.BlockSpec((D, D), lambda i: (0, 0))
_b_spec = pl.BlockSpec((1, D), lambda i: (0, 0))
_h_spec = pl.BlockSpec((BLK, D), lambda i: (i, 0))
_p_spec = pl.BlockSpec((NC, BLK, D), lambda i: (0, i, 0))

_mlp1 = pl.pallas_call(
    _mlp1_body,
    grid=(N // BLK,),
    in_specs=[_h_spec, _p_spec, _w_spec, _b_spec, _w_spec, _b_spec],
    out_specs=_h_spec,
    out_shape=jax.ShapeDtypeStruct((N, D), jnp.float32),
)

_mlp2 = pl.pallas_call(
    _mlp2_body,
    grid=(N // BLK,),
    in_specs=[_h_spec, _p_spec, _w_spec, _b_spec, _w_spec, _b_spec,
              _w_spec, _b_spec],
    out_specs=pl.BlockSpec((1, D), lambda i: (0, 0)),
    out_shape=jax.ShapeDtypeStruct((1, D), jnp.float32),
    scratch_shapes=[pltpu.VMEM((1, D), jnp.float32)],
)


def kernel(x, edge_index, W1a, b1a, W1b, b1b, W2a, b2a, W2b, b2b, Wlin, blin):
    pad = E_PAD - E
    src = jnp.concatenate(
        [edge_index[0], jnp.zeros((pad,), jnp.int32)]).reshape(E_PAD // CH, CH)
    trash = N + (jnp.arange(pad, dtype=jnp.int32) % (N_PAD - N))
    dst = jnp.concatenate(
        [edge_index[1], trash]).reshape(E_PAD // CH, CH)
    zeros = jnp.zeros((128, D), jnp.float32)
    b1a2, b1b2 = b1a.reshape(1, D), b1b.reshape(1, D)
    b2a2, b2b2 = b2a.reshape(1, D), b2b.reshape(1, D)
    blin2 = blin.reshape(1, D)

    p1 = _sc_agg(x, src, dst, zeros)
    h1 = _mlp1(x, p1, W1a, b1a2, W1b, b1b2)
    p2 = _sc_agg(h1, src, dst, zeros)
    return _mlp2(h1, p2, W2a, b2a2, W2b, b2b2, Wlin, blin2)


# speedup vs baseline: 3.1065x; 3.1033x over previous
import functools

import jax
import jax.numpy as jnp
from jax import lax
from jax.experimental import pallas as pl
from jax.experimental.pallas import tpu as pltpu
from jax.experimental.pallas import tpu_sc as plsc

N = 10000
E = 320000
D = 128

NC = 2
NS = 16
CH = 128
NCH = 80
TPW = NCH * CH
E_PAD = NC * NS * TPW
N_PAD = 10240
RPT = N_PAD // NS


PH = NCH // 2


def _sc_agg_body(h_hbm, src_hbm, dst_hbm, zeros_hbm, out_hbm,
                 idx_s, idx_d, rows, agg_sh, sg0, sg1, ss0, ss1, si):
    cid = lax.axis_index("c")
    sid = lax.axis_index("s")
    tile = cid * NS + sid
    sg = [sg0, sg1]
    ss = [ss0, ss1]

    r0 = sid * RPT
    for k in range(RPT // 128):
        pltpu.sync_copy(zeros_hbm, agg_sh.at[pl.ds(r0 + k * 128, 128)])

    plsc.subcore_barrier()

    def g_start(j, b):
        pltpu.async_copy(h_hbm.at[idx_s.at[j]], rows.at[b], sg[b])

    def g_wait(b):
        pltpu.make_async_copy(h_hbm.at[idx_s.at[0]], rows.at[b], sg[b]).wait()

    def s_start(i, b):
        pltpu.async_copy(rows.at[b], agg_sh.at[idx_d.at[i]], ss[b], add=True)

    def s_wait(b):
        pltpu.make_async_copy(rows.at[b], agg_sh.at[idx_d.at[0]], ss[b]).wait()

    for p in range(2):
        base = tile * NCH + p * PH
        pltpu.async_copy(src_hbm.at[pl.ds(base, PH)], idx_s, si)
        pltpu.async_copy(dst_hbm.at[pl.ds(base, PH)], idx_d, si)
        pltpu.make_async_copy(src_hbm.at[pl.ds(base, PH)], idx_s, si).wait()
        pltpu.make_async_copy(dst_hbm.at[pl.ds(base, PH)], idx_d, si).wait()

        g_start(0, 0)
        g_wait(0); s_start(0, 0); g_start(1, 1)

        def pair(j, carry):
            c = 2 * j + 1
            g_wait(1); s_start(c, 1); s_wait(0); g_start(c + 1, 0)
            g_wait(0); s_start(c + 1, 0); s_wait(1); g_start(c + 2, 1)
            return carry

        lax.fori_loop(0, (PH - 2) // 2, pair, 0)

        g_wait(1); s_start(PH - 1, 1)
        s_wait(0)
        s_wait(1)

    plsc.subcore_barrier()

    pltpu.sync_copy(agg_sh.at[pl.ds(r0, RPT)],
                    out_hbm.at[cid].at[pl.ds(r0, RPT)])


_sc_agg = functools.partial(
    pl.kernel,
    out_type=jax.ShapeDtypeStruct((NC, N_PAD, D), jnp.float32),
    mesh=plsc.VectorSubcoreMesh(
        core_axis_name="c", subcore_axis_name="s", num_cores=NC,
        num_subcores=NS),
    scratch_types=[
        pltpu.VMEM((PH, CH), jnp.int32),
        pltpu.VMEM((PH, CH), jnp.int32),
        pltpu.VMEM((2, CH, D), jnp.float32),
        pltpu.VMEM_SHARED((N_PAD, D), jnp.float32),
        pltpu.SemaphoreType.DMA, pltpu.SemaphoreType.DMA,
        pltpu.SemaphoreType.DMA, pltpu.SemaphoreType.DMA,
        pltpu.SemaphoreType.DMA,
    ],
)(_sc_agg_body)


BLK = 1000


def _mlp1_body(h_ref, p_ref, wa_ref, ba_ref, wb_ref, bb_ref, o_ref):
    z = h_ref[...] + p_ref[0] + p_ref[1]
    z = jnp.maximum(
        jnp.dot(z, wa_ref[...], preferred_element_type=jnp.float32)
        + ba_ref[...], 0.0)
    z = jnp.dot(z, wb_ref[...], preferred_element_type=jnp.float32) \
        + bb_ref[...]
    o_ref[...] = jnp.maximum(z, 0.0)


def _mlp2_body(h_ref, p_ref, wa_ref, ba_ref, wb_ref, bb_ref,
               wl_ref, bl_ref, o_ref, acc_ref):
    i = pl.program_id(0)
    z = h_ref[...] + p_ref[0] + p_ref[1]
    z = jnp.maximum(
        jnp.dot(z, wa_ref[...], preferred_element_type=jnp.float32)
        + ba_ref[...], 0.0)
    z = jnp.dot(z, wb_ref[...], preferred_element_type=jnp.float32) \
        + bb_ref[...]
    col = jnp.sum(jnp.maximum(z, 0.0), axis=0, keepdims=True)

    @pl.when(i == 0)
    def _():
        acc_ref[...] = col

    @pl.when(i > 0)
    def _():
        acc_ref[...] = acc_ref[...] + col

    @pl.when(i == pl.num_programs(0) - 1)
    def _():
        o_ref[...] = jnp.dot(acc_ref[...], wl_ref[...],
                             preferred_element_type=jnp.float32) + bl_ref[...]


_w_spec = pl.BlockSpec((D, D), lambda i: (0, 0))
_b_spec = pl.BlockSpec((1, D), lambda i: (0, 0))
_h_spec = pl.BlockSpec((BLK, D), lambda i: (i, 0))
_p_spec = pl.BlockSpec((NC, BLK, D), lambda i: (0, i, 0))

_mlp1 = pl.pallas_call(
    _mlp1_body,
    grid=(N // BLK,),
    in_specs=[_h_spec, _p_spec, _w_spec, _b_spec, _w_spec, _b_spec],
    out_specs=_h_spec,
    out_shape=jax.ShapeDtypeStruct((N, D), jnp.float32),
)

_mlp2 = pl.pallas_call(
    _mlp2_body,
    grid=(N // BLK,),
    in_specs=[_h_spec, _p_spec, _w_spec, _b_spec, _w_spec, _b_spec,
              _w_spec, _b_spec],
    out_specs=pl.BlockSpec((1, D), lambda i: (0, 0)),
    out_shape=jax.ShapeDtypeStruct((1, D), jnp.float32),
    scratch_shapes=[pltpu.VMEM((1, D), jnp.float32)],
)


def kernel(x, edge_index, W1a, b1a, W1b, b1b, W2a, b2a, W2b, b2b, Wlin, blin):
    pad = E_PAD - E
    src_pad = jnp.arange(pad, dtype=jnp.int32) % N
    src = jnp.concatenate(
        [edge_index[0], src_pad]).reshape(E_PAD // CH, CH)
    trash = N + (jnp.arange(pad, dtype=jnp.int32) % (N_PAD - N))
    dst = jnp.concatenate(
        [edge_index[1], trash]).reshape(E_PAD // CH, CH)
    zeros = jnp.zeros((128, D), jnp.float32)
    b1a2, b1b2 = b1a.reshape(1, D), b1b.reshape(1, D)
    b2a2, b2b2 = b2a.reshape(1, D), b2b.reshape(1, D)
    blin2 = blin.reshape(1, D)

    p1 = _sc_agg(x, src, dst, zeros)
    h1 = _mlp1(x, p1, W1a, b1a2, W1b, b1b2)
    p2 = _sc_agg(h1, src, dst, zeros)
    return _mlp2(h1, p2, W2a, b2a2, W2b, b2b2, Wlin, blin2)


# speedup vs baseline: 3.5112x; 1.1303x over previous
import functools

import jax
import jax.numpy as jnp
from jax import lax
from jax.experimental import pallas as pl
from jax.experimental.pallas import tpu as pltpu
from jax.experimental.pallas import tpu_sc as plsc

N = 10000
E = 320000
D = 128

NC = 2
NS = 16
CH = 64
NCH = 160
TPW = NCH * CH
E_PAD = NC * NS * TPW
N_PAD = 10240
RPT = N_PAD // NS

NBUF = 4
PH = NCH // 4


def _sc_agg_body(h_hbm, src_hbm, dst_hbm, zeros_hbm, out_hbm,
                 idx_s, idx_d, rows, agg_sh,
                 sg0, sg1, sg2, sg3, ss0, ss1, ss2, ss3, si):
    cid = lax.axis_index("c")
    sid = lax.axis_index("s")
    tile = cid * NS + sid
    sg = [sg0, sg1, sg2, sg3]
    ss = [ss0, ss1, ss2, ss3]

    r0 = sid * RPT
    for k in range(RPT // 128):
        pltpu.sync_copy(zeros_hbm, agg_sh.at[pl.ds(r0 + k * 128, 128)])

    plsc.subcore_barrier()

    def g_start(j, b):
        pltpu.async_copy(h_hbm.at[idx_s.at[j]], rows.at[b], sg[b])

    def g_wait(b):
        pltpu.make_async_copy(h_hbm.at[idx_s.at[0]], rows.at[b], sg[b]).wait()

    def s_start(i, b):
        pltpu.async_copy(rows.at[b], agg_sh.at[idx_d.at[i]], ss[b], add=True)

    def s_wait(b):
        pltpu.make_async_copy(rows.at[b], agg_sh.at[idx_d.at[0]], ss[b]).wait()

    for p in range(4):
        base = tile * NCH + p * PH
        pltpu.async_copy(src_hbm.at[pl.ds(base, PH)], idx_s, si)
        pltpu.async_copy(dst_hbm.at[pl.ds(base, PH)], idx_d, si)
        pltpu.make_async_copy(src_hbm.at[pl.ds(base, PH)], idx_s, si).wait()
        pltpu.make_async_copy(dst_hbm.at[pl.ds(base, PH)], idx_d, si).wait()

        g_start(0, 0)
        g_start(1, 1)
        g_start(2, 2)

        g_wait(0); s_start(0, 0); g_start(3, 3)
        g_wait(1); s_start(1, 1); s_wait(0); g_start(4, 0)
        g_wait(2); s_start(2, 2); s_wait(1); g_start(5, 1)
        g_wait(3); s_start(3, 3); s_wait(2); g_start(6, 2)

        def group(g, carry):
            for b in range(NBUF):
                i = g * NBUF + b
                bn = (b + 3) % NBUF
                g_wait(b)
                s_start(i, b)
                s_wait(bn)
                g_start(i + 3, bn)
            return carry

        lax.fori_loop(1, PH // NBUF - 1, group, 0)

        g_wait(0); s_start(PH - 4, 0); s_wait(3); g_start(PH - 1, 3)
        g_wait(1); s_start(PH - 3, 1)
        g_wait(2); s_start(PH - 2, 2)
        g_wait(3); s_start(PH - 1, 3)
        for b in range(NBUF):
            s_wait(b)

    plsc.subcore_barrier()

    pltpu.sync_copy(agg_sh.at[pl.ds(r0, RPT)],
                    out_hbm.at[cid].at[pl.ds(r0, RPT)])


_sc_agg = functools.partial(
    pl.kernel,
    out_type=jax.ShapeDtypeStruct((NC, N_PAD, D), jnp.float32),
    mesh=plsc.VectorSubcoreMesh(
        core_axis_name="c", subcore_axis_name="s", num_cores=NC,
        num_subcores=NS),
    scratch_types=[
        pltpu.VMEM((PH, CH), jnp.int32),
        pltpu.VMEM((PH, CH), jnp.int32),
        pltpu.VMEM((NBUF, CH, D), jnp.float32),
        pltpu.VMEM_SHARED((N_PAD, D), jnp.float32),
        pltpu.SemaphoreType.DMA, pltpu.SemaphoreType.DMA,
        pltpu.SemaphoreType.DMA, pltpu.SemaphoreType.DMA,
        pltpu.SemaphoreType.DMA, pltpu.SemaphoreType.DMA,
        pltpu.SemaphoreType.DMA, pltpu.SemaphoreType.DMA,
        pltpu.SemaphoreType.DMA,
    ],
)(_sc_agg_body)


BLK = 1000


def _mlp1_body(h_ref, p_ref, wa_ref, ba_ref, wb_ref, bb_ref, o_ref):
    z = h_ref[...] + p_ref[0] + p_ref[1]
    z = jnp.maximum(
        jnp.dot(z, wa_ref[...], preferred_element_type=jnp.float32)
        + ba_ref[...], 0.0)
    z = jnp.dot(z, wb_ref[...], preferred_element_type=jnp.float32) \
        + bb_ref[...]
    o_ref[...] = jnp.maximum(z, 0.0)


def _mlp2_body(h_ref, p_ref, wa_ref, ba_ref, wb_ref, bb_ref,
               wl_ref, bl_ref, o_ref, acc_ref):
    i = pl.program_id(0)
    z = h_ref[...] + p_ref[0] + p_ref[1]
    z = jnp.maximum(
        jnp.dot(z, wa_ref[...], preferred_element_type=jnp.float32)
        + ba_ref[...], 0.0)
    z = jnp.dot(z, wb_ref[...], preferred_element_type=jnp.float32) \
        + bb_ref[...]
    col = jnp.sum(jnp.maximum(z, 0.0), axis=0, keepdims=True)

    @pl.when(i == 0)
    def _():
        acc_ref[...] = col

    @pl.when(i > 0)
    def _():
        acc_ref[...] = acc_ref[...] + col

    @pl.when(i == pl.num_programs(0) - 1)
    def _():
        o_ref[...] = jnp.dot(acc_ref[...], wl_ref[...],
                             preferred_element_type=jnp.float32) + bl_ref[...]


_w_spec = pl.BlockSpec((D, D), lambda i: (0, 0))
_b_spec = pl.BlockSpec((1, D), lambda i: (0, 0))
_h_spec = pl.BlockSpec((BLK, D), lambda i: (i, 0))
_p_spec = pl.BlockSpec((NC, BLK, D), lambda i: (0, i, 0))

_mlp1 = pl.pallas_call(
    _mlp1_body,
    grid=(N // BLK,),
    in_specs=[_h_spec, _p_spec, _w_spec, _b_spec, _w_spec, _b_spec],
    out_specs=_h_spec,
    out_shape=jax.ShapeDtypeStruct((N, D), jnp.float32),
)

_mlp2 = pl.pallas_call(
    _mlp2_body,
    grid=(N // BLK,),
    in_specs=[_h_spec, _p_spec, _w_spec, _b_spec, _w_spec, _b_spec,
              _w_spec, _b_spec],
    out_specs=pl.BlockSpec((1, D), lambda i: (0, 0)),
    out_shape=jax.ShapeDtypeStruct((1, D), jnp.float32),
    scratch_shapes=[pltpu.VMEM((1, D), jnp.float32)],
)


def kernel(x, edge_index, W1a, b1a, W1b, b1b, W2a, b2a, W2b, b2b, Wlin, blin):
    pad = E_PAD - E
    src_pad = jnp.arange(pad, dtype=jnp.int32) % N
    src = jnp.concatenate(
        [edge_index[0], src_pad]).reshape(E_PAD // CH, CH)
    trash = N + (jnp.arange(pad, dtype=jnp.int32) % (N_PAD - N))
    dst = jnp.concatenate(
        [edge_index[1], trash]).reshape(E_PAD // CH, CH)
    zeros = jnp.zeros((128, D), jnp.float32)
    b1a2, b1b2 = b1a.reshape(1, D), b1b.reshape(1, D)
    b2a2, b2b2 = b2a.reshape(1, D), b2b.reshape(1, D)
    blin2 = blin.reshape(1, D)

    p1 = _sc_agg(x, src, dst, zeros)
    h1 = _mlp1(x, p1, W1a, b1a2, W1b, b1b2)
    p2 = _sc_agg(h1, src, dst, zeros)
    return _mlp2(h1, p2, W2a, b2a2, W2b, b2b2, Wlin, blin2)


# speedup vs baseline: 3.5112x; 1.0000x over previous
import functools

import jax
import jax.numpy as jnp
from jax import lax
from jax.experimental import pallas as pl
from jax.experimental.pallas import tpu as pltpu
from jax.experimental.pallas import tpu_sc as plsc

N = 10000
E = 320000
D = 128

NC = 2
NS = 16
CH = 64
NCH = 160
TPW = NCH * CH
E_PAD = NC * NS * TPW
N_PAD = 10240
RPT = N_PAD // NS

NBUF = 4
PH = NCH // 4


def _sc_agg_body(h_hbm, src_hbm, dst_hbm, zeros_hbm, out_hbm,
                 idx_s, idx_d, rows, agg_sh, *sems):
    cid = lax.axis_index("c")
    sid = lax.axis_index("s")
    tile = cid * NS + sid
    sg = list(sems[:NBUF])
    ss = list(sems[NBUF:2 * NBUF])
    si = sems[2 * NBUF]

    r0 = sid * RPT
    for k in range(RPT // 128):
        pltpu.sync_copy(zeros_hbm, agg_sh.at[pl.ds(r0 + k * 128, 128)])

    plsc.subcore_barrier()

    def g_start(j, b):
        pltpu.async_copy(h_hbm.at[idx_s.at[j]], rows.at[b], sg[b])

    def g_wait(b):
        pltpu.make_async_copy(h_hbm.at[idx_s.at[0]], rows.at[b], sg[b]).wait()

    def s_start(i, b):
        pltpu.async_copy(rows.at[b], agg_sh.at[idx_d.at[i]], ss[b], add=True)

    def s_wait(b):
        pltpu.make_async_copy(rows.at[b], agg_sh.at[idx_d.at[0]], ss[b]).wait()

    for p in range(4):
        base = tile * NCH + p * PH
        pltpu.async_copy(src_hbm.at[pl.ds(base, PH)], idx_s, si)
        pltpu.async_copy(dst_hbm.at[pl.ds(base, PH)], idx_d, si)
        pltpu.make_async_copy(src_hbm.at[pl.ds(base, PH)], idx_s, si).wait()
        pltpu.make_async_copy(dst_hbm.at[pl.ds(base, PH)], idx_d, si).wait()

        for b in range(NBUF - 1):
            g_start(b, b)

        g_wait(0); s_start(0, 0); g_start(NBUF - 1, NBUF - 1)
        for b in range(1, NBUF):
            g_wait(b); s_start(b, b); s_wait(b - 1)
            g_start(b + NBUF - 1, b - 1)

        def group(g, carry):
            for b in range(NBUF):
                i = g * NBUF + b
                bn = (b + NBUF - 1) % NBUF
                g_wait(b)
                s_start(i, b)
                s_wait(bn)
                g_start(i + NBUF - 1, bn)
            return carry

        lax.fori_loop(1, PH // NBUF - 1, group, 0)

        g_wait(0); s_start(PH - NBUF, 0); s_wait(NBUF - 1)
        g_start(PH - 1, NBUF - 1)
        for b in range(1, NBUF):
            g_wait(b); s_start(PH - NBUF + b, b)
        for b in range(NBUF):
            s_wait(b)

    plsc.subcore_barrier()

    pltpu.sync_copy(agg_sh.at[pl.ds(r0, RPT)],
                    out_hbm.at[cid].at[pl.ds(r0, RPT)])


_sc_agg = functools.partial(
    pl.kernel,
    out_type=jax.ShapeDtypeStruct((NC, N_PAD, D), jnp.float32),
    mesh=plsc.VectorSubcoreMesh(
        core_axis_name="c", subcore_axis_name="s", num_cores=NC,
        num_subcores=NS),
    scratch_types=[
        pltpu.VMEM((PH, CH), jnp.int32),
        pltpu.VMEM((PH, CH), jnp.int32),
        pltpu.VMEM((NBUF, CH, D), jnp.float32),
        pltpu.VMEM_SHARED((N_PAD, D), jnp.float32),
    ] + [pltpu.SemaphoreType.DMA] * (2 * NBUF + 1),
)(_sc_agg_body)


BLK = 1000


def _mlp1_body(h_ref, p_ref, wa_ref, ba_ref, wb_ref, bb_ref, o_ref):
    z = h_ref[...] + p_ref[0] + p_ref[1]
    z = jnp.maximum(
        jnp.dot(z, wa_ref[...], preferred_element_type=jnp.float32)
        + ba_ref[...], 0.0)
    z = jnp.dot(z, wb_ref[...], preferred_element_type=jnp.float32) \
        + bb_ref[...]
    o_ref[...] = jnp.maximum(z, 0.0)


def _mlp2_body(h_ref, p_ref, wa_ref, ba_ref, wb_ref, bb_ref,
               wl_ref, bl_ref, o_ref, acc_ref):
    i = pl.program_id(0)
    z = h_ref[...] + p_ref[0] + p_ref[1]
    z = jnp.maximum(
        jnp.dot(z, wa_ref[...], preferred_element_type=jnp.float32)
        + ba_ref[...], 0.0)
    z = jnp.dot(z, wb_ref[...], preferred_element_type=jnp.float32) \
        + bb_ref[...]
    col = jnp.sum(jnp.maximum(z, 0.0), axis=0, keepdims=True)

    @pl.when(i == 0)
    def _():
        acc_ref[...] = col

    @pl.when(i > 0)
    def _():
        acc_ref[...] = acc_ref[...] + col

    @pl.when(i == pl.num_programs(0) - 1)
    def _():
        o_ref[...] = jnp.dot(acc_ref[...], wl_ref[...],
                             preferred_element_type=jnp.float32) + bl_ref[...]


_w_spec = pl.BlockSpec((D, D), lambda i: (0, 0))
_b_spec = pl.BlockSpec((1, D), lambda i: (0, 0))
_h_spec = pl.BlockSpec((BLK, D), lambda i: (i, 0))
_p_spec = pl.BlockSpec((NC, BLK, D), lambda i: (0, i, 0))

_mlp1 = pl.pallas_call(
    _mlp1_body,
    grid=(N // BLK,),
    in_specs=[_h_spec, _p_spec, _w_spec, _b_spec, _w_spec, _b_spec],
    out_specs=_h_spec,
    out_shape=jax.ShapeDtypeStruct((N, D), jnp.float32),
)

_mlp2 = pl.pallas_call(
    _mlp2_body,
    grid=(N // BLK,),
    in_specs=[_h_spec, _p_spec, _w_spec, _b_spec, _w_spec, _b_spec,
              _w_spec, _b_spec],
    out_specs=pl.BlockSpec((1, D), lambda i: (0, 0)),
    out_shape=jax.ShapeDtypeStruct((1, D), jnp.float32),
    scratch_shapes=[pltpu.VMEM((1, D), jnp.float32)],
)


def kernel(x, edge_index, W1a, b1a, W1b, b1b, W2a, b2a, W2b, b2b, Wlin, blin):
    pad = E_PAD - E
    src_pad = jnp.arange(pad, dtype=jnp.int32) % N
    src = jnp.concatenate(
        [edge_index[0], src_pad]).reshape(E_PAD // CH, CH)
    trash = N + (jnp.arange(pad, dtype=jnp.int32) % (N_PAD - N))
    dst = jnp.concatenate(
        [edge_index[1], trash]).reshape(E_PAD // CH, CH)
    zeros = jnp.zeros((128, D), jnp.float32)
    b1a2, b1b2 = b1a.reshape(1, D), b1b.reshape(1, D)
    b2a2, b2b2 = b2a.reshape(1, D), b2b.reshape(1, D)
    blin2 = blin.reshape(1, D)

    p1 = _sc_agg(x, src, dst, zeros)
    h1 = _mlp1(x, p1, W1a, b1a2, W1b, b1b2)
    p2 = _sc_agg(h1, src, dst, zeros)
    return _mlp2(h1, p2, W2a, b2a2, W2b, b2b2, Wlin, blin2)
